# baseline (speedup 1.0000x reference)
import functools

import jax
import jax.numpy as jnp
from jax import lax
from jax.experimental import pallas as pl
from jax.experimental.pallas import tpu as pltpu
from jax.experimental.pallas import tpu_sc as plsc

VOCAB = 1000000
BATCH = 16384
UNITS = 10

_NC = 2
_NS = 16
_NW = _NC * _NS
_BPW = BATCH // _NW
_ICH = 128
_KCH = _BPW // _ICH
_L = 16
_CPG = _ICH // _L
_OPW = _BPW * UNITS

_mesh = plsc.VectorSubcoreMesh(
    core_axis_name="c", subcore_axis_name="s", num_cores=_NC, num_subcores=_NS
)


def _affine_pats(wsc, bsc):
    l16 = lax.iota(jnp.int32, _L)
    terms, bpats = [], []
    for m in range(UNITS):
        n = l16 + 16 * m
        row = lax.shift_right_logical(n * 6554, 16)
        col = n - row * 10
        wp = jnp.zeros((_L,), jnp.float32)
        bp = jnp.zeros((_L,), jnp.float32)
        for j in range(UNITS):
            mk = jnp.where(col == j, jnp.float32(1), jnp.float32(0))
            wp = wp + mk * wsc[j]
            bp = bp + mk * bsc[j]
        srcs = sorted({(16 * m + l) // 10 for l in range(_L)})
        terms.append(
            [(a, wp * jnp.where(row == a, jnp.float32(1), jnp.float32(0)))
             for a in srcs]
        )
        bpats.append(bp)
    return terms, bpats


@functools.partial(
    pl.kernel,
    out_type=jax.ShapeDtypeStruct((_NW, _OPW), jnp.float32),
    mesh=_mesh,
    scratch_types=[
        pltpu.VMEM((_KCH, _ICH), jnp.int32),
        pltpu.VMEM((_BPW,), jnp.int32),
        pltpu.VMEM((2, _L), jnp.float32),
        pltpu.VMEM((_OPW,), jnp.float32),
        pltpu.SemaphoreType.DMA,
        pltpu.SemaphoreType.DMA,
        pltpu.SemaphoreType.DMA,
        pltpu.SemaphoreType.DMA,
        pltpu.SemaphoreType.DMA,
    ],
)
def _lookup_affine(
    table_h, idx_h, w_h, b_h, out_h, idx_v, ids_v, wb_v, out_v, wb_sem, *sems
):
    wid = lax.axis_index("s") * _NC + lax.axis_index("c")
    wcp = pltpu.async_copy(w_h, wb_v.at[0, pl.ds(0, UNITS)], wb_sem)
    bcp = pltpu.async_copy(b_h, wb_v.at[1, pl.ds(0, UNITS)], wb_sem)
    pltpu.sync_copy(idx_h.at[wid], idx_v)
    copies = [
        pltpu.async_copy(
            table_h.at[idx_v.at[k]], ids_v.at[pl.ds(k * _ICH, _ICH)], sems[k]
        )
        for k in range(_KCH)
    ]
    wcp.wait()
    bcp.wait()
    wrow = wb_v[0]
    brow = wb_v[1]
    terms, bpats = _affine_pats(
        [wrow[j] for j in range(UNITS)], [brow[j] for j in range(UNITS)]
    )
    for k in range(_KCH):
        copies[k].wait()
        for cc in range(_CPG):
            c = k * _CPG + cc
            idf = ids_v[pl.ds(c * _L, _L)].astype(jnp.float32)
            s = [idf[l] for l in range(_L)]
            for m in range(UNITS):
                a0, w0 = terms[m][0]
                acc = s[a0] * w0 + bpats[m]
                for a, wm in terms[m][1:]:
                    acc = acc + s[a] * wm
                out_v[pl.ds(c * UNITS * _L + m * _L, _L)] = acc
    pltpu.sync_copy(out_v, out_h.at[wid])


def kernel(inputs, lookup_table, W, b):
    idx = inputs.reshape(-1).astype(jnp.int32).reshape(_NW, _KCH, _ICH)
    out = _lookup_affine(
        lookup_table, idx, W.reshape(UNITS).astype(jnp.float32),
        b.astype(jnp.float32)
    )
    return out.reshape(BATCH, UNITS)


# speedup vs baseline: 1.5302x; 1.5302x over previous
import functools

import jax
import jax.numpy as jnp
from jax import lax
from jax.experimental import pallas as pl
from jax.experimental.pallas import tpu as pltpu
from jax.experimental.pallas import tpu_sc as plsc

VOCAB = 1000000
BATCH = 16384
UNITS = 10

_NC = 2
_NS = 16
_NW = _NC * _NS
_BPW = BATCH // _NW
_ICH = 128
_KCH = _BPW // _ICH
_LANES = 16

_mesh = plsc.VectorSubcoreMesh(
    core_axis_name="c", subcore_axis_name="s", num_cores=_NC, num_subcores=_NS
)


@functools.partial(
    pl.kernel,
    out_type=jax.ShapeDtypeStruct((_NW, UNITS, _BPW), jnp.float32),
    mesh=_mesh,
    scratch_types=[
        pltpu.VMEM((_KCH, _ICH), jnp.int32),
        pltpu.VMEM((_BPW,), jnp.int32),
        pltpu.VMEM((2, _LANES), jnp.float32),
        pltpu.VMEM((UNITS, _BPW), jnp.float32),
        pltpu.SemaphoreType.DMA,
    ],
)
def _lookup_affine(table_h, idx_h, wb_h, out_h, idx_v, ids_v, wb_v, out_v, sem):
    wid = lax.axis_index("s") * _NC + lax.axis_index("c")
    pltpu.sync_copy(idx_h.at[wid], idx_v)
    pltpu.sync_copy(wb_h, wb_v)
    copies = [
        pltpu.async_copy(
            table_h.at[idx_v.at[k]], ids_v.at[pl.ds(k * _ICH, _ICH)], sem
        )
        for k in range(_KCH)
    ]
    for c in copies:
        c.wait()
    wrow = wb_v[0]
    brow = wb_v[1]
    ws = [wrow[j] for j in range(UNITS)]
    bs = [brow[j] for j in range(UNITS)]
    for i in range(_BPW // _LANES):
        v = ids_v[pl.ds(i * _LANES, _LANES)].astype(jnp.float32)
        for j in range(UNITS):
            out_v[j, pl.ds(i * _LANES, _LANES)] = v * ws[j] + bs[j]
    pltpu.sync_copy(out_v, out_h.at[wid])


def kernel(inputs, lookup_table, W, b):
    idx = inputs.reshape(-1).astype(jnp.int32).reshape(_NW, _KCH, _ICH)
    wb = jnp.zeros((2, _LANES), jnp.float32)
    wb = wb.at[0, :UNITS].set(W[0].astype(jnp.float32))
    wb = wb.at[1, :UNITS].set(b.astype(jnp.float32))
    out = _lookup_affine(lookup_table, idx, wb)
    return out.transpose(0, 2, 1).reshape(BATCH, UNITS)
